# baseline (speedup 1.0000x reference)
import functools

import jax
import jax.numpy as jnp
from jax import lax
from jax.experimental import pallas as pl
from jax.experimental.pallas import tpu as pltpu
from jax.experimental.pallas import tpu_sc as plsc

B = 16384
NWF = 100000
DF = 32
H = 100000
D = 128
PRIME_PAIRS = ((10007, 3), (10009, 7), (10037, 11), (10039, 13), (10061, 17))
NT = 5

NC = 2
NS = 16
NTILES = NC * NS
IDS_PER_TILE = 2 * B // NTILES
CHUNK = 128
NCHUNK = IDS_PER_TILE // CHUNK

LANES = 16
VPC = CHUNK // LANES

NBUF = 4


def _tile_coords():
    wid = lax.axis_index("s") * NC + lax.axis_index("c")
    return wid // NS, wid % NS


def _gather_pipelined(src, idx_v, i, out, side, base, rows, gs, os):
    def move_group(half, _):
        cbase = half * NBUF
        g = []
        for k in range(NBUF):
            g.append(pltpu.async_copy(
                src.at[idx_v.at[i, cbase + k]], rows[k], gs[k]))
        o = []
        for k in range(NBUF):
            g[k].wait()
            o.append(pltpu.async_copy(
                rows[k],
                out.at[pl.ds(base + (cbase + k) * CHUNK, CHUNK)],
                os[k]))
        for k in range(NBUF):
            o[k].wait()
        return _

    lax.fori_loop(0, NCHUNK // NBUF, move_group, 0)


def _sc_tables(ids_hbm, t0, t1, t2, t3, t4, s_out,
               ids_v, idx_v, r0, r1, r2, r3, gs0, gs1, gs2, gs3,
               os0, os1, os2, os3):
    tables = (t0, t1, t2, t3, t4)
    rows = (r0, r1, r2, r3)
    gs = (gs0, gs1, gs2, gs3)
    os = (os0, os1, os2, os3)
    side, tile = _tile_coords()
    base = tile * IDS_PER_TILE
    pltpu.sync_copy(ids_hbm.at[side, tile], ids_v)

    for i in range(NT):
        p, o = PRIME_PAIRS[i]

        def hash_chunk(c, _, p=p, o=o, i=i):
            for j in range(VPC):
                v = ids_v[c, pl.ds(j * LANES, LANES)]
                idx_v[i, c, pl.ds(j * LANES, LANES)] = (v * p + o) % H
            return _

        lax.fori_loop(0, NCHUNK, hash_chunk, 0)

    for i, src in enumerate(tables):
        _gather_pipelined(src, idx_v, i, s_out.at[i, side], side, base,
                          rows, gs, os)


def _sc_wf(ids_hbm, wf4_hbm, w_out,
           ids_v, idx_v, r0, r1, r2, r3, gs0, gs1, gs2, gs3,
           os0, os1, os2, os3):
    rows = (r0, r1, r2, r3)
    gs = (gs0, gs1, gs2, gs3)
    os = (os0, os1, os2, os3)
    side, tile = _tile_coords()
    base = tile * IDS_PER_TILE
    pltpu.sync_copy(ids_hbm.at[side, tile], ids_v)

    def hash_chunk(c, _):
        for j in range(VPC):
            v = ids_v[c, pl.ds(j * LANES, LANES)]
            idx_v[0, c, pl.ds(j * LANES, LANES)] = v >> 2
        return _

    lax.fori_loop(0, NCHUNK, hash_chunk, 0)
    _gather_pipelined(wf4_hbm, idx_v, 0, w_out.at[side], side, base,
                      rows, gs, os)


def _sc_scratch(n_idx):
    return [
        pltpu.VMEM((NCHUNK, CHUNK), jnp.int32),
        pltpu.VMEM((n_idx, NCHUNK, CHUNK), jnp.int32),
    ] + [pltpu.VMEM((CHUNK, D), jnp.float32)] * NBUF \
      + [pltpu.SemaphoreType.DMA] * (2 * NBUF)


@functools.cache
def _sc_tables_call():
    return pl.kernel(
        _sc_tables,
        out_type=jax.ShapeDtypeStruct((NT, 2, B, D), jnp.float32),
        mesh=plsc.VectorSubcoreMesh(core_axis_name="c", subcore_axis_name="s"),
        scratch_types=_sc_scratch(NT),
    )


@functools.cache
def _sc_wf_call():
    return pl.kernel(
        _sc_wf,
        out_type=jax.ShapeDtypeStruct((2, B, D), jnp.float32),
        mesh=plsc.VectorSubcoreMesh(core_axis_name="c", subcore_axis_name="s"),
        scratch_types=_sc_scratch(1),
    )


P = 1024
INV_BN = 1.0 / (1.0 + 1e-5) ** 0.5


def _row_dot(a, b):
    return jnp.sum(a * b, axis=1, keepdims=True)


def _cos_col(a, b):
    num = _row_dot(a, b)
    den = jnp.maximum(jnp.sqrt(_row_dot(a, a)) * jnp.sqrt(_row_dot(b, b)), 1e-8)
    return num / den


def _dense_from_packed(packed, m, dW, db):
    feat = jnp.zeros((packed.shape[0], DF), jnp.float32)
    for k in range(4):
        mask = (m == k).astype(jnp.float32)
        feat = feat + packed[:, 32 * k:32 * (k + 1)] * mask
    return jnp.dot(feat, dW, preferred_element_type=jnp.float32) + db


def _tc_body(s_ref, w_ref, i1_ref, i2_ref, dW_ref, db_ref, cW_ref, cb_ref,
             g_ref, be_ref, lw_ref, lb_ref,
             score_ref, e1_ref, e2_ref):
    dW = dW_ref[...]
    db = db_ref[...]
    cb = cb_ref[...]

    d1 = _dense_from_packed(w_ref[0], i1_ref[...] & 3, dW, db)
    d2 = _dense_from_packed(w_ref[1], i2_ref[...] & 3, dW, db)
    e1 = cb + jnp.dot(d1, cW_ref[0], preferred_element_type=jnp.float32)
    e2 = cb + jnp.dot(d2, cW_ref[0], preferred_element_type=jnp.float32)
    for i in range(NT):
        e1 = e1 + jnp.dot(s_ref[i, 0], cW_ref[i + 1],
                          preferred_element_type=jnp.float32)
        e2 = e2 + jnp.dot(s_ref[i, 1], cW_ref[i + 1],
                          preferred_element_type=jnp.float32)
    e1_ref[...] = e1
    e2_ref[...] = e2

    cols = [_cos_col(e1, e2), _cos_col(d1, d2)]
    for i in range(NT):
        cols.append(_cos_col(s_ref[i, 0], s_ref[i, 1]))
    cos = jnp.concatenate(cols, axis=1)
    xn = cos * INV_BN * g_ref[...] + be_ref[...]
    score_ref[...] = jnp.sum(xn * lw_ref[...], axis=1, keepdims=True) + lb_ref[...]


def _tc_call(s, w, ids1, ids2, dW, db, cW3, cb, g, be, lw, lb):
    nblk = B // P
    return pl.pallas_call(
        _tc_body,
        grid=(nblk,),
        in_specs=[
            pl.BlockSpec((NT, 2, P, D), lambda b: (0, 0, b, 0)),
            pl.BlockSpec((2, P, D), lambda b: (0, b, 0)),
            pl.BlockSpec((P, 1), lambda b: (b, 0)),
            pl.BlockSpec((P, 1), lambda b: (b, 0)),
            pl.BlockSpec((DF, D), lambda b: (0, 0)),
            pl.BlockSpec((1, D), lambda b: (0, 0)),
            pl.BlockSpec((NT + 1, D, D), lambda b: (0, 0, 0)),
            pl.BlockSpec((1, D), lambda b: (0, 0)),
            pl.BlockSpec((1, 7), lambda b: (0, 0)),
            pl.BlockSpec((1, 7), lambda b: (0, 0)),
            pl.BlockSpec((1, 7), lambda b: (0, 0)),
            pl.BlockSpec((1, 1), lambda b: (0, 0)),
        ],
        out_specs=[
            pl.BlockSpec((P, 1), lambda b: (b, 0)),
            pl.BlockSpec((P, D), lambda b: (b, 0)),
            pl.BlockSpec((P, D), lambda b: (b, 0)),
        ],
        out_shape=[
            jax.ShapeDtypeStruct((B, 1), jnp.float32),
            jax.ShapeDtypeStruct((B, D), jnp.float32),
            jax.ShapeDtypeStruct((B, D), jnp.float32),
        ],
    )(s, w, ids1, ids2, dW, db, cW3, cb, g, be, lw, lb)


def kernel(work_pairs, work_features, table_0, table_1, table_2, table_3, table_4,
           dense_W, dense_b, comb_W, comb_b, bn_gamma, bn_beta, lin_W, lin_b):
    wp = work_pairs.astype(jnp.int32)
    ids = wp.T.reshape(2, NS, NCHUNK, CHUNK)
    wf4 = work_features.reshape(NWF // 4, 4 * DF)
    s = _sc_tables_call()(ids, table_0, table_1, table_2, table_3, table_4)
    w = _sc_wf_call()(ids, wf4)
    score2, e1, e2 = _tc_call(
        s, w, wp[:, 0:1], wp[:, 1:2],
        dense_W, dense_b.reshape(1, D), comb_W.reshape(NT + 1, D, D),
        comb_b.reshape(1, D),
        bn_gamma.reshape(1, 7), bn_beta.reshape(1, 7),
        lin_W.reshape(1, 7), lin_b.reshape(1, 1),
    )
    return (score2[:, 0], e1, e2)


# speedup vs baseline: 1.0747x; 1.0747x over previous
import functools

import jax
import jax.numpy as jnp
from jax import lax
from jax.experimental import pallas as pl
from jax.experimental.pallas import tpu as pltpu
from jax.experimental.pallas import tpu_sc as plsc

B = 16384
NWF = 100000
DF = 32
H = 100000
D = 128
PRIME_PAIRS = ((10007, 3), (10009, 7), (10037, 11), (10039, 13), (10061, 17))
NT = 5

NC = 2
NS = 16
NTILES = NC * NS
IDS_PER_TILE = 2 * B // NTILES
CHUNK = 128
NCHUNK = IDS_PER_TILE // CHUNK

LANES = 16
VPC = CHUNK // LANES

NBUF = 4


def _tile_coords():
    wid = lax.axis_index("s") * NC + lax.axis_index("c")
    return wid // NS, wid % NS


def _gather_pipelined(src, idx_v, i, out, side, base, rows, gs, os):
    def move_group(half, _):
        cbase = half * NBUF
        g = []
        for k in range(NBUF):
            g.append(pltpu.async_copy(
                src.at[idx_v.at[i, cbase + k]], rows[k], gs[k]))
        o = []
        for k in range(NBUF):
            g[k].wait()
            o.append(pltpu.async_copy(
                rows[k],
                out.at[pl.ds(base + (cbase + k) * CHUNK, CHUNK)],
                os[k]))
        for k in range(NBUF):
            o[k].wait()
        return _

    lax.fori_loop(0, NCHUNK // NBUF, move_group, 0)


def _sc_tables(ids_hbm, t0, t1, t2, t3, t4, s_out,
               ids_v, idx_v, r0, r1, r2, r3, gs0, gs1, gs2, gs3,
               os0, os1, os2, os3):
    tables = (t0, t1, t2, t3, t4)
    rows = (r0, r1, r2, r3)
    gs = (gs0, gs1, gs2, gs3)
    os = (os0, os1, os2, os3)
    side, tile = _tile_coords()
    base = tile * IDS_PER_TILE
    pltpu.sync_copy(ids_hbm.at[side, tile], ids_v)

    for i in range(NT):
        p, o = PRIME_PAIRS[i]

        def hash_chunk(c, _, p=p, o=o, i=i):
            for j in range(VPC):
                v = ids_v[c, pl.ds(j * LANES, LANES)]
                idx_v[i, c, pl.ds(j * LANES, LANES)] = (v * p + o) % H
            return _

        lax.fori_loop(0, NCHUNK, hash_chunk, 0)

    for i, src in enumerate(tables):
        _gather_pipelined(src, idx_v, i, s_out.at[i, side], side, base,
                          rows, gs, os)


def _sc_wf(ids_hbm, wf4_hbm, w_out,
           ids_v, idx_v, r0, r1, r2, r3, gs0, gs1, gs2, gs3,
           os0, os1, os2, os3):
    rows = (r0, r1, r2, r3)
    gs = (gs0, gs1, gs2, gs3)
    os = (os0, os1, os2, os3)
    side, tile = _tile_coords()
    base = tile * IDS_PER_TILE
    pltpu.sync_copy(ids_hbm.at[side, tile], ids_v)

    def hash_chunk(c, _):
        for j in range(VPC):
            v = ids_v[c, pl.ds(j * LANES, LANES)]
            idx_v[0, c, pl.ds(j * LANES, LANES)] = v >> 2
        return _

    lax.fori_loop(0, NCHUNK, hash_chunk, 0)
    _gather_pipelined(wf4_hbm, idx_v, 0, w_out.at[side], side, base,
                      rows, gs, os)


def _sc_scratch(n_idx):
    return [
        pltpu.VMEM((NCHUNK, CHUNK), jnp.int32),
        pltpu.VMEM((n_idx, NCHUNK, CHUNK), jnp.int32),
    ] + [pltpu.VMEM((CHUNK, D), jnp.float32)] * NBUF \
      + [pltpu.SemaphoreType.DMA] * (2 * NBUF)


@functools.cache
def _sc_tables_call():
    return pl.kernel(
        _sc_tables,
        out_type=jax.ShapeDtypeStruct((NT, 2, B, D), jnp.float32),
        mesh=plsc.VectorSubcoreMesh(core_axis_name="c", subcore_axis_name="s"),
        scratch_types=_sc_scratch(NT),
    )


@functools.cache
def _sc_wf_call():
    return pl.kernel(
        _sc_wf,
        out_type=jax.ShapeDtypeStruct((2, B, D), jnp.float32),
        mesh=plsc.VectorSubcoreMesh(core_axis_name="c", subcore_axis_name="s"),
        scratch_types=_sc_scratch(1),
    )


P = 1024
INV_BN = 1.0 / (1.0 + 1e-5) ** 0.5


def _row_dot(a, b):
    return jnp.sum(a * b, axis=1, keepdims=True)


def _cos_col(a, b):
    num = _row_dot(a, b)
    den = jnp.maximum(jnp.sqrt(_row_dot(a, a)) * jnp.sqrt(_row_dot(b, b)), 1e-8)
    return num / den


def _mm(a, b):
    return jnp.dot(a.astype(jnp.bfloat16), b.astype(jnp.bfloat16),
                   preferred_element_type=jnp.float32)


def _dense_from_packed(packed, m, dW, db):
    acc = db
    for k in range(4):
        mask = (m == k).astype(jnp.float32)
        mk = jnp.concatenate(
            ([jnp.zeros((32 * k, D), jnp.float32)] if k else [])
            + [dW]
            + ([jnp.zeros((96 - 32 * k, D), jnp.float32)] if k < 3 else []),
            axis=0)
        acc = acc + _mm(packed * mask, mk)
    return acc


def _tc_body(s_ref, w_ref, i1_ref, i2_ref, dW_ref, db_ref, cW_ref, cb_ref,
             g_ref, be_ref, lw_ref, lb_ref,
             score_ref, e1_ref, e2_ref):
    dW = dW_ref[...]
    db = db_ref[...]
    cb = cb_ref[...]

    d1 = _dense_from_packed(w_ref[0], i1_ref[...] & 3, dW, db)
    d2 = _dense_from_packed(w_ref[1], i2_ref[...] & 3, dW, db)
    e1 = cb + _mm(d1, cW_ref[0])
    e2 = cb + _mm(d2, cW_ref[0])
    for i in range(NT):
        e1 = e1 + _mm(s_ref[i, 0], cW_ref[i + 1])
        e2 = e2 + _mm(s_ref[i, 1], cW_ref[i + 1])
    e1_ref[...] = e1
    e2_ref[...] = e2

    cols = [_cos_col(e1, e2), _cos_col(d1, d2)]
    for i in range(NT):
        cols.append(_cos_col(s_ref[i, 0], s_ref[i, 1]))
    cos = jnp.concatenate(cols, axis=1)
    xn = cos * INV_BN * g_ref[...] + be_ref[...]
    score_ref[...] = jnp.sum(xn * lw_ref[...], axis=1, keepdims=True) + lb_ref[...]


def _tc_call(s, w, ids1, ids2, dW, db, cW3, cb, g, be, lw, lb):
    nblk = B // P
    return pl.pallas_call(
        _tc_body,
        grid=(nblk,),
        in_specs=[
            pl.BlockSpec((NT, 2, P, D), lambda b: (0, 0, b, 0)),
            pl.BlockSpec((2, P, D), lambda b: (0, b, 0)),
            pl.BlockSpec((P, 1), lambda b: (b, 0)),
            pl.BlockSpec((P, 1), lambda b: (b, 0)),
            pl.BlockSpec((DF, D), lambda b: (0, 0)),
            pl.BlockSpec((1, D), lambda b: (0, 0)),
            pl.BlockSpec((NT + 1, D, D), lambda b: (0, 0, 0)),
            pl.BlockSpec((1, D), lambda b: (0, 0)),
            pl.BlockSpec((1, 7), lambda b: (0, 0)),
            pl.BlockSpec((1, 7), lambda b: (0, 0)),
            pl.BlockSpec((1, 7), lambda b: (0, 0)),
            pl.BlockSpec((1, 1), lambda b: (0, 0)),
        ],
        out_specs=[
            pl.BlockSpec((P, 1), lambda b: (b, 0)),
            pl.BlockSpec((P, D), lambda b: (b, 0)),
            pl.BlockSpec((P, D), lambda b: (b, 0)),
        ],
        out_shape=[
            jax.ShapeDtypeStruct((B, 1), jnp.float32),
            jax.ShapeDtypeStruct((B, D), jnp.float32),
            jax.ShapeDtypeStruct((B, D), jnp.float32),
        ],
    )(s, w, ids1, ids2, dW, db, cW3, cb, g, be, lw, lb)


def kernel(work_pairs, work_features, table_0, table_1, table_2, table_3, table_4,
           dense_W, dense_b, comb_W, comb_b, bn_gamma, bn_beta, lin_W, lin_b):
    wp = work_pairs.astype(jnp.int32)
    ids = wp.T.reshape(2, NS, NCHUNK, CHUNK)
    wf4 = work_features.reshape(NWF // 4, 4 * DF)
    s = _sc_tables_call()(ids, table_0, table_1, table_2, table_3, table_4)
    w = _sc_wf_call()(ids, wf4)
    score2, e1, e2 = _tc_call(
        s, w, wp[:, 0:1], wp[:, 1:2],
        dense_W, dense_b.reshape(1, D), comb_W.reshape(NT + 1, D, D),
        comb_b.reshape(1, D),
        bn_gamma.reshape(1, 7), bn_beta.reshape(1, 7),
        lin_W.reshape(1, 7), lin_b.reshape(1, 1),
    )
    return (score2[:, 0], e1, e2)


# speedup vs baseline: 1.1023x; 1.0256x over previous
import functools

import jax
import jax.numpy as jnp
from jax import lax
from jax.experimental import pallas as pl
from jax.experimental.pallas import tpu as pltpu
from jax.experimental.pallas import tpu_sc as plsc

B = 16384
NWF = 100000
DF = 32
H = 100000
D = 128
PRIME_PAIRS = ((10007, 3), (10009, 7), (10037, 11), (10039, 13), (10061, 17))
NT = 5

NC = 2
NS = 16
NTILES = NC * NS
IDS_PER_TILE = 2 * B // NTILES
CHUNK = 128
NCHUNK = IDS_PER_TILE // CHUNK

LANES = 16
VPC = CHUNK // LANES

NBUF = 4
HB = B // 2
NCH = NCHUNK // 2


def _tile_coords():
    wid = lax.axis_index("s") * NC + lax.axis_index("c")
    return wid // NS, wid % NS


def _gather_group(src, idx_row, out, rows, gs, os):
    g = []
    for k in range(NBUF):
        g.append(pltpu.async_copy(src.at[idx_row.at[k]], rows[k], gs[k]))
    o = []
    for k in range(NBUF):
        g[k].wait()
        o.append(pltpu.async_copy(
            rows[k], out.at[pl.ds(k * CHUNK, CHUNK)], os[k]))
    for k in range(NBUF):
        o[k].wait()


def _hash_chunks(ids_v, idx_v, i, p, o):
    def hash_chunk(c, _):
        for j in range(VPC):
            v = ids_v[c, pl.ds(j * LANES, LANES)]
            if p is None:
                idx_v[i, c, pl.ds(j * LANES, LANES)] = v >> 2
            else:
                idx_v[i, c, pl.ds(j * LANES, LANES)] = (v * p + o) % H
        return _

    lax.fori_loop(0, NCH, hash_chunk, 0)


def _sc_tables_half(h):
    def body(ids_hbm, t0, t1, t2, t3, t4, s_out,
             ids_v, idx_v, r0, r1, r2, r3, gs0, gs1, gs2, gs3,
             os0, os1, os2, os3):
        tables = (t0, t1, t2, t3, t4)
        rows = (r0, r1, r2, r3)
        gs = (gs0, gs1, gs2, gs3)
        os = (os0, os1, os2, os3)
        side, tile = _tile_coords()
        base = tile * (NCH * CHUNK)
        pltpu.sync_copy(ids_hbm.at[side, h, tile], ids_v)
        for i in range(NT):
            p, o = PRIME_PAIRS[i]
            _hash_chunks(ids_v, idx_v, i, p, o)
        for i, src in enumerate(tables):
            _gather_group(src, idx_v.at[i],
                          s_out.at[i, side, pl.ds(base, NCH * CHUNK)],
                          rows, gs, os)

    return body


def _sc_wf(ids_hbm, wf4_hbm, w_out,
           ids_v, idx_v, r0, r1, r2, r3, gs0, gs1, gs2, gs3,
           os0, os1, os2, os3):
    rows = (r0, r1, r2, r3)
    gs = (gs0, gs1, gs2, gs3)
    os = (os0, os1, os2, os3)
    side, tile = _tile_coords()
    for h in range(2):
        base = h * HB + tile * (NCH * CHUNK)
        pltpu.sync_copy(ids_hbm.at[side, h, tile], ids_v)
        _hash_chunks(ids_v, idx_v, 0, None, None)
        _gather_group(wf4_hbm, idx_v.at[0],
                      w_out.at[side, pl.ds(base, NCH * CHUNK)],
                      rows, gs, os)


def _sc_scratch(n_idx):
    return [
        pltpu.VMEM((NCH, CHUNK), jnp.int32),
        pltpu.VMEM((n_idx, NCH, CHUNK), jnp.int32),
    ] + [pltpu.VMEM((CHUNK, D), jnp.float32)] * NBUF \
      + [pltpu.SemaphoreType.DMA] * (2 * NBUF)


@functools.cache
def _sc_tables_call(h):
    return pl.kernel(
        _sc_tables_half(h),
        out_type=jax.ShapeDtypeStruct((NT, 2, HB, D), jnp.float32),
        mesh=plsc.VectorSubcoreMesh(core_axis_name="c", subcore_axis_name="s"),
        scratch_types=_sc_scratch(NT),
    )


@functools.cache
def _sc_wf_call():
    return pl.kernel(
        _sc_wf,
        out_type=jax.ShapeDtypeStruct((2, B, D), jnp.float32),
        mesh=plsc.VectorSubcoreMesh(core_axis_name="c", subcore_axis_name="s"),
        scratch_types=_sc_scratch(1),
    )


P = 1024
INV_BN = 1.0 / (1.0 + 1e-5) ** 0.5


def _row_dot(a, b):
    return jnp.sum(a * b, axis=1, keepdims=True)


def _cos_col(a, b):
    num = _row_dot(a, b)
    den = jnp.maximum(jnp.sqrt(_row_dot(a, a)) * jnp.sqrt(_row_dot(b, b)), 1e-8)
    return num / den


def _mm(a, b):
    return jnp.dot(a, b, preferred_element_type=jnp.float32)


def _dense_from_packed(packed, m, dW, db):
    acc = db
    for k in range(4):
        mask = (m == k).astype(jnp.float32)
        mk = jnp.concatenate(
            ([jnp.zeros((32 * k, D), jnp.float32)] if k else [])
            + [dW]
            + ([jnp.zeros((96 - 32 * k, D), jnp.float32)] if k < 3 else []),
            axis=0)
        acc = acc + _mm(packed * mask, mk)
    return acc


def _tc_body(s_ref, w_ref, i1_ref, i2_ref, dW_ref, db_ref, cW_ref, cb_ref,
             g_ref, be_ref, lw_ref, lb_ref,
             score_ref, e1_ref, e2_ref):
    dW = dW_ref[...]
    db = db_ref[...]
    cb = cb_ref[...]

    d1 = _dense_from_packed(w_ref[0], i1_ref[...] & 3, dW, db)
    d2 = _dense_from_packed(w_ref[1], i2_ref[...] & 3, dW, db)
    e1 = cb + _mm(d1, cW_ref[0])
    e2 = cb + _mm(d2, cW_ref[0])
    for i in range(NT):
        e1 = e1 + _mm(s_ref[i, 0], cW_ref[i + 1])
        e2 = e2 + _mm(s_ref[i, 1], cW_ref[i + 1])
    e1_ref[...] = e1
    e2_ref[...] = e2

    cols = [_cos_col(e1, e2), _cos_col(d1, d2)]
    for i in range(NT):
        cols.append(_cos_col(s_ref[i, 0], s_ref[i, 1]))
    cos = jnp.concatenate(cols, axis=1)
    xn = cos * INV_BN * g_ref[...] + be_ref[...]
    score_ref[...] = jnp.sum(xn * lw_ref[...], axis=1, keepdims=True) + lb_ref[...]


def _tc_call(s, w, ids1, ids2, dW, db, cW3, cb, g, be, lw, lb, blk0):
    nblk = HB // P
    return pl.pallas_call(
        _tc_body,
        grid=(nblk,),
        in_specs=[
            pl.BlockSpec((NT, 2, P, D), lambda b: (0, 0, b, 0)),
            pl.BlockSpec((2, P, D), lambda b: (0, b + blk0, 0)),
            pl.BlockSpec((P, 1), lambda b: (b + blk0, 0)),
            pl.BlockSpec((P, 1), lambda b: (b + blk0, 0)),
            pl.BlockSpec((DF, D), lambda b: (0, 0)),
            pl.BlockSpec((1, D), lambda b: (0, 0)),
            pl.BlockSpec((NT + 1, D, D), lambda b: (0, 0, 0)),
            pl.BlockSpec((1, D), lambda b: (0, 0)),
            pl.BlockSpec((1, 7), lambda b: (0, 0)),
            pl.BlockSpec((1, 7), lambda b: (0, 0)),
            pl.BlockSpec((1, 7), lambda b: (0, 0)),
            pl.BlockSpec((1, 1), lambda b: (0, 0)),
        ],
        out_specs=[
            pl.BlockSpec((P, 1), lambda b: (b, 0)),
            pl.BlockSpec((P, D), lambda b: (b, 0)),
            pl.BlockSpec((P, D), lambda b: (b, 0)),
        ],
        out_shape=[
            jax.ShapeDtypeStruct((HB, 1), jnp.float32),
            jax.ShapeDtypeStruct((HB, D), jnp.float32),
            jax.ShapeDtypeStruct((HB, D), jnp.float32),
        ],
    )(s, w, ids1, ids2, dW, db, cW3, cb, g, be, lw, lb)


def kernel(work_pairs, work_features, table_0, table_1, table_2, table_3, table_4,
           dense_W, dense_b, comb_W, comb_b, bn_gamma, bn_beta, lin_W, lin_b):
    wp = work_pairs.astype(jnp.int32)
    ids = wp.T.reshape(2, 2, NS, NCH, CHUNK)
    wf4 = work_features.reshape(NWF // 4, 4 * DF)
    sA = _sc_tables_call(0)(ids, table_0, table_1, table_2, table_3, table_4)
    w = _sc_wf_call()(ids, wf4)
    sB = _sc_tables_call(1)(ids, table_0, table_1, table_2, table_3, table_4)
    weights = (dense_W, dense_b.reshape(1, D), comb_W.reshape(NT + 1, D, D),
               comb_b.reshape(1, D), bn_gamma.reshape(1, 7),
               bn_beta.reshape(1, 7), lin_W.reshape(1, 7), lin_b.reshape(1, 1))
    ids1, ids2 = wp[:, 0:1], wp[:, 1:2]
    scA, e1A, e2A = _tc_call(sA, w, ids1, ids2, *weights, blk0=0)
    scB, e1B, e2B = _tc_call(sB, w, ids1, ids2, *weights, blk0=HB // P)
    score2 = jnp.concatenate([scA, scB], axis=0)
    e1 = jnp.concatenate([e1A, e1B], axis=0)
    e2 = jnp.concatenate([e2A, e2B], axis=0)
    return (score2[:, 0], e1, e2)
